# baseline (speedup 1.0000x reference)
import functools

import jax
import jax.numpy as jnp
from jax.experimental import pallas as pl
from jax.experimental.pallas import tpu as pltpu


def _knn_body(n_steps, bn, q_ref, mb_ref, lane_ref,
              scores_ref, img_ref, idx_ref, vmin_ref, vidx_ref):
    step = pl.program_id(0)

    @pl.when(step == 0)
    def _init():
        vmin_ref[...] = jnp.full(vmin_ref.shape, jnp.inf, jnp.float32)
        vidx_ref[...] = jnp.zeros(vidx_ref.shape, jnp.int32)

    q = q_ref[...]
    mb = mb_ref[...]
    acc = jax.lax.dot_general(
        q * -2.0, mb, (((1,), (1,)), ((), ())),
        preferred_element_type=jnp.float32)
    q_sq = jnp.sum(q * q, axis=1, keepdims=True)
    m_sq = jnp.sum(mb * mb, axis=1)
    d2 = (q_sq + acc) + m_sq[None, :]

    bmin = jnp.min(d2, axis=1, keepdims=True)
    bidx = jnp.argmin(d2, axis=1)[:, None] + step * bn

    better = bmin < vmin_ref[...]
    vidx_ref[...] = jnp.where(better, bidx, vidx_ref[...])
    vmin_ref[...] = jnp.where(better, bmin, vmin_ref[...])

    @pl.when(step == n_steps - 1)
    def _finish():
        s = jnp.sqrt(jnp.maximum(vmin_ref[...], 0.0) + 1e-12)
        scores_ref[...] = s
        img_ref[...] = jnp.max(s).reshape(1, 1)
        idx_ref[...] = vidx_ref[...]


def kernel(queries, memory_bank, k):
    Q, D = queries.shape
    N = memory_bank.shape[0]
    BN = 5000
    n_steps = N // BN

    scores, img, idx = pl.pallas_call(
        functools.partial(_knn_body, n_steps, BN),
        grid=(n_steps,),
        in_specs=[
            pl.BlockSpec((Q, D), lambda i: (0, 0)),
            pl.BlockSpec((BN, D), lambda i: (i, 0)),
            pl.BlockSpec((1, BN), lambda i: (0, 0)),
        ],
        out_specs=[
            pl.BlockSpec((Q, 1), lambda i: (0, 0)),
            pl.BlockSpec((1, 1), lambda i: (0, 0)),
            pl.BlockSpec((Q, 1), lambda i: (0, 0)),
        ],
        out_shape=[
            jax.ShapeDtypeStruct((Q, 1), jnp.float32),
            jax.ShapeDtypeStruct((1, 1), jnp.float32),
            jax.ShapeDtypeStruct((Q, 1), jnp.int32),
        ],
        scratch_shapes=[
            pltpu.VMEM((Q, 1), jnp.float32),
            pltpu.VMEM((Q, 1), jnp.int32),
        ],
    )(queries, memory_bank, jnp.arange(BN, dtype=jnp.float32)[None, :])

    kf = jnp.asarray(k, jnp.float32)
    patch_scores = scores[:, 0] / kf
    image_score = img[0, 0] / kf
    return (patch_scores, image_score, idx)


# speedup vs baseline: 1.1599x; 1.1599x over previous
import functools

import jax
import jax.numpy as jnp
from jax.experimental import pallas as pl
from jax.experimental.pallas import tpu as pltpu


def _knn_body(n_steps, bn, q_ref, mb_ref, lane_ref,
              scores_ref, img_ref, idx_ref, vmin_ref, vidx_ref):
    step = pl.program_id(0)

    @pl.when(step == 0)
    def _init():
        vmin_ref[...] = jnp.full(vmin_ref.shape, jnp.inf, jnp.float32)
        vidx_ref[...] = jnp.zeros(vidx_ref.shape, jnp.int32)

    q = q_ref[...]
    mb = mb_ref[...]
    acc = jax.lax.dot_general(
        q * -2.0, mb, (((1,), (1,)), ((), ())),
        preferred_element_type=jnp.float32)
    q_sq = jnp.sum(q * q, axis=1, keepdims=True)
    m_sq = jnp.sum(mb * mb, axis=1)
    d2 = (q_sq + acc) + m_sq[None, :]

    bmin = jnp.min(d2, axis=1, keepdims=True)
    bidx_f = jnp.min(jnp.where(d2 == bmin, lane_ref[...], jnp.float32(bn)),
                     axis=1, keepdims=True)
    bidx = bidx_f.astype(jnp.int32) + step * bn

    better = bmin < vmin_ref[...]
    vidx_ref[...] = jnp.where(better, bidx, vidx_ref[...])
    vmin_ref[...] = jnp.where(better, bmin, vmin_ref[...])

    @pl.when(step == n_steps - 1)
    def _finish():
        s = jnp.sqrt(jnp.maximum(vmin_ref[...], 0.0) + 1e-12)
        scores_ref[...] = s
        img_ref[...] = jnp.max(s).reshape(1, 1)
        idx_ref[...] = vidx_ref[...]


def kernel(queries, memory_bank, k):
    Q, D = queries.shape
    N = memory_bank.shape[0]
    BN = 5000
    n_steps = N // BN

    scores, img, idx = pl.pallas_call(
        functools.partial(_knn_body, n_steps, BN),
        grid=(n_steps,),
        in_specs=[
            pl.BlockSpec((Q, D), lambda i: (0, 0)),
            pl.BlockSpec((BN, D), lambda i: (i, 0)),
            pl.BlockSpec((1, BN), lambda i: (0, 0)),
        ],
        out_specs=[
            pl.BlockSpec((Q, 1), lambda i: (0, 0)),
            pl.BlockSpec((1, 1), lambda i: (0, 0)),
            pl.BlockSpec((Q, 1), lambda i: (0, 0)),
        ],
        out_shape=[
            jax.ShapeDtypeStruct((Q, 1), jnp.float32),
            jax.ShapeDtypeStruct((1, 1), jnp.float32),
            jax.ShapeDtypeStruct((Q, 1), jnp.int32),
        ],
        scratch_shapes=[
            pltpu.VMEM((Q, 1), jnp.float32),
            pltpu.VMEM((Q, 1), jnp.int32),
        ],
    )(queries, memory_bank, jnp.arange(BN, dtype=jnp.float32)[None, :])

    kf = jnp.asarray(k, jnp.float32)
    patch_scores = scores[:, 0] / kf
    image_score = img[0, 0] / kf
    return (patch_scores, image_score, idx)
